# baseline (speedup 1.0000x reference)
import functools

import jax
import jax.numpy as jnp
import numpy as np
from jax import lax
from jax.experimental import pallas as pl
from jax.experimental.pallas import tpu as pltpu
from jax.experimental.pallas import tpu_sc as plsc

_ANCHORS = np.array([[116.0, 90.0], [156.0, 198.0], [373.0, 326.0]], dtype=np.float32)
_NUM_CLASSES = 80
_IMG_DIM = 608.0
_P = 304


def _sigmoid(v):
    return 1.0 / (1.0 + jnp.exp(-v))


def _sc_decode(x_hbm, o_hbm, ib0, ib1, ob0, ob1, is0, is1, os0, os1, *, B, G, stride):
    nA = _ANCHORS.shape[0]
    nch = _NUM_CLASSES + 5
    GG = G * G
    n_chunks = GG // _P
    n_tasks = B * nA * n_chunks

    info = plsc.get_sparse_core_info()
    NC, NS = info.num_cores, info.num_subcores
    NW = NC * NS
    wid = lax.axis_index("s") * NC + lax.axis_index("c")

    t_lo = (wid * n_tasks) // NW
    t_hi = ((wid + 1) * n_tasks) // NW
    n_mine = t_hi - t_lo

    ibufs = (ib0, ib1)
    obufs = (ob0, ob1)
    isems = (is0, is1)
    osems = (os0, os1)

    lane = lax.iota(jnp.int32, 16)
    lane_f = lane.astype(jnp.float32)
    del lane_f

    def task_coords(t):
        b = t // (nA * n_chunks)
        r = t % (nA * n_chunks)
        a = r // n_chunks
        k = r % n_chunks
        return b, a, k * _P

    def start_in(t, slot):
        b, a, p0 = task_coords(t)
        return pltpu.async_copy(
            x_hbm.at[b, a, :, pl.ds(p0, _P)], ibufs[slot], isems[slot]
        )

    def wait_in(slot):
        pltpu.make_async_copy(
            x_hbm.at[0, 0, :, pl.ds(0, _P)], ibufs[slot], isems[slot]
        ).wait()

    def start_out(t, slot):
        b, a, p0 = task_coords(t)
        return pltpu.async_copy(
            obufs[slot], o_hbm.at[b, pl.ds(a * GG + p0, _P), :], osems[slot]
        )

    def wait_out(slot):
        pltpu.make_async_copy(
            obufs[slot], o_hbm.at[0, pl.ds(0, _P), :], osems[slot]
        ).wait()

    def compute(t, slot):
        ib = ibufs[slot]
        ob = obufs[slot]
        _, _, p0 = task_coords(t)
        a = (t // n_chunks) % nA
        af = a.astype(jnp.float32)
        aw = jnp.where(
            af == 0.0,
            _ANCHORS[0, 0],
            jnp.where(af == 1.0, _ANCHORS[1, 0], _ANCHORS[2, 0]),
        )
        ah = jnp.where(
            af == 0.0,
            _ANCHORS[0, 1],
            jnp.where(af == 1.0, _ANCHORS[1, 1], _ANCHORS[2, 1]),
        )

        def col_idx(j):
            return lane + j * 16

        def body01(j, _):
            cols = col_idx(j)
            p = cols + p0
            gy = p // G
            gx = p - gy * G
            v0 = ib[0, pl.ds(j * 16, 16)]
            v1 = ib[1, pl.ds(j * 16, 16)]
            r0 = (_sigmoid(v0) + gx.astype(jnp.float32)) * stride
            r1 = (_sigmoid(v1) + gy.astype(jnp.float32)) * stride
            plsc.store_scatter(ob, [cols, jnp.full((16,), 0, jnp.int32)], r0)
            plsc.store_scatter(ob, [cols, jnp.full((16,), 1, jnp.int32)], r1)
            return 0

        lax.fori_loop(0, _P // 16, body01, 0, unroll=True)

        def body23(j, _):
            cols = col_idx(j)
            v2 = ib[2, pl.ds(j * 16, 16)]
            v3 = ib[3, pl.ds(j * 16, 16)]
            r2 = jnp.exp(v2) * aw
            r3 = jnp.exp(v3) * ah
            plsc.store_scatter(ob, [cols, jnp.full((16,), 2, jnp.int32)], r2)
            plsc.store_scatter(ob, [cols, jnp.full((16,), 3, jnp.int32)], r3)
            return 0

        lax.fori_loop(0, _P // 16, body23, 0, unroll=True)

        def body_c(c, _):
            def body_j(j, _):
                cols = col_idx(j)
                v = ib[c, pl.ds(j * 16, 16)]
                plsc.store_scatter(ob, [cols, jnp.full((16,), 0, jnp.int32) + c], _sigmoid(v))
                return 0

            lax.fori_loop(0, _P // 16, body_j, 0, unroll=True)
            return 0

        lax.fori_loop(4, nch, body_c, 0)

    @pl.when(n_mine > 0)
    def _prime():
        start_in(t_lo, 0)

    def pair(g, _):
        for slot in (0, 1):
            i = 2 * g + slot

            @pl.when(i < n_mine)
            def _step(i=i, slot=slot):
                t = t_lo + i

                @pl.when(i + 1 < n_mine)
                def _pref():
                    start_in(t + 1, 1 - slot)

                wait_in(slot)

                @pl.when(i >= 2)
                def _wout():
                    wait_out(slot)

                compute(t, slot)
                start_out(t, slot)

        return 0

    lax.fori_loop(0, (n_mine + 1) // 2, pair, 0)

    for s in (0, 1):
        @pl.when((n_mine >= 1) & (lax.rem(n_mine - 1, 2) == s))
        def _d1(s=s):
            wait_out(s)

        @pl.when((n_mine >= 2) & (lax.rem(n_mine - 2, 2) == s))
        def _d2(s=s):
            wait_out(s)


def kernel(x):
    B = x.shape[0]
    G = x.shape[2]
    nA = _ANCHORS.shape[0]
    nch = _NUM_CLASSES + 5
    GG = G * G
    stride = _IMG_DIM / G

    xr = x.reshape(B, nA, nch, GG)

    mesh = plsc.VectorSubcoreMesh(core_axis_name="c", subcore_axis_name="s")
    sc_fn = functools.partial(_sc_decode, B=B, G=G, stride=stride)
    out = pl.kernel(
        sc_fn,
        mesh=mesh,
        out_type=jax.ShapeDtypeStruct((B, nA * GG, nch), jnp.float32),
        scratch_types=[
            pltpu.VMEM((nch, _P), jnp.float32),
            pltpu.VMEM((nch, _P), jnp.float32),
            pltpu.VMEM((_P, nch), jnp.float32),
            pltpu.VMEM((_P, nch), jnp.float32),
            pltpu.SemaphoreType.DMA,
            pltpu.SemaphoreType.DMA,
            pltpu.SemaphoreType.DMA,
            pltpu.SemaphoreType.DMA,
        ],
        compiler_params=pltpu.CompilerParams(
            use_tc_tiling_on_sc=False, needs_layout_passes=False
        ),
    )(xr)

    return out


# speedup vs baseline: 7.7361x; 7.7361x over previous
import functools

import jax
import jax.numpy as jnp
import numpy as np
from jax.experimental import pallas as pl
from jax.experimental.pallas import tpu as pltpu

_ANCHORS = np.array([[116.0, 90.0], [156.0, 198.0], [373.0, 326.0]], dtype=np.float32)
_NUM_CLASSES = 80
_IMG_DIM = 608.0


def _decode_kernel(x_ref, o_ref, *, G, stride, anchors):
    a = pl.program_id(1)
    X = x_ref[0]
    nch = _NUM_CLASSES + 5

    sig = jax.nn.sigmoid(X)

    top = X[0:8]
    row8 = jax.lax.broadcasted_iota(jnp.int32, (8, G, G), 0)
    gy = jax.lax.broadcasted_iota(jnp.int32, (1, G, G), 1).astype(jnp.float32)
    gx = jax.lax.broadcasted_iota(jnp.int32, (1, G, G), 2).astype(jnp.float32)

    ex = jnp.exp(top)
    sig8 = sig[0:8]

    aw = jnp.where(a == 0, anchors[0, 0], jnp.where(a == 1, anchors[1, 0], anchors[2, 0]))
    ah = jnp.where(a == 0, anchors[0, 1], jnp.where(a == 1, anchors[1, 1], anchors[2, 1]))

    base = jnp.where((row8 == 2) | (row8 == 3), ex, sig8)
    add = jnp.where(row8 == 0, gx, jnp.where(row8 == 1, gy, 0.0))
    scale = jnp.where(
        row8 < 2, stride, jnp.where(row8 == 2, aw, jnp.where(row8 == 3, ah, 1.0))
    )
    top_out = (base + add) * scale

    y = jnp.concatenate([top_out, sig[8:]], axis=0)

    for gyi in range(G):
        o_ref[0, gyi * G : (gyi + 1) * G, :] = y[:, gyi, :].T


def kernel(x):
    B = x.shape[0]
    G = x.shape[2]
    nA = _ANCHORS.shape[0]
    nch = _NUM_CLASSES + 5
    GG = G * G
    stride = _IMG_DIM / G

    out = pl.pallas_call(
        functools.partial(_decode_kernel, G=G, stride=stride, anchors=_ANCHORS),
        grid=(B, nA),
        in_specs=[pl.BlockSpec((1, nch, G, G), lambda b, a: (b, a, 0, 0))],
        out_specs=pl.BlockSpec((1, GG, nch), lambda b, a: (b, a, 0)),
        out_shape=jax.ShapeDtypeStruct((B, nA * GG, nch), jnp.float32),
        compiler_params=pltpu.CompilerParams(
            dimension_semantics=("parallel", "arbitrary"),
        ),
    )(x)

    return out


# speedup vs baseline: 8.3903x; 1.0846x over previous
import functools

import jax
import jax.numpy as jnp
import numpy as np
from jax.experimental import pallas as pl
from jax.experimental.pallas import tpu as pltpu

_ANCHORS = np.array([[116.0, 90.0], [156.0, 198.0], [373.0, 326.0]], dtype=np.float32)
_NUM_CLASSES = 80
_IMG_DIM = 608.0
_NBUF = 3


def _decode_kernel(x_hbm, o_hbm, ibuf, obuf, isem, osem, *, G, stride, anchors, nsteps):
    nA = anchors.shape[0]
    GG = G * G
    nch = _NUM_CLASSES + 5
    i = pl.program_id(0)
    slot = jax.lax.rem(i, _NBUF)
    b = jax.lax.div(i, nA)
    a = jax.lax.rem(i, nA)

    @pl.when(i == 0)
    def _warmup():
        for k in range(_NBUF):
            pltpu.make_async_copy(
                x_hbm.at[k // nA, pl.ds((k % nA) * nch, nch)], ibuf.at[k], isem.at[k]
            ).start()

    pltpu.make_async_copy(
        x_hbm.at[0, pl.ds(0, nch)], ibuf.at[slot], isem.at[slot]
    ).wait()

    @pl.when(i >= _NBUF)
    def _wait_out():
        pltpu.make_async_copy(
            obuf.at[slot], o_hbm.at[0, pl.ds(0, GG), :], osem.at[slot]
        ).wait()

    X = ibuf[slot]

    sig = jax.nn.sigmoid(X)

    top = X[0:8]
    row8 = jax.lax.broadcasted_iota(jnp.int32, (8, G, G), 0)
    gy = jax.lax.broadcasted_iota(jnp.int32, (1, G, G), 1).astype(jnp.float32)
    gx = jax.lax.broadcasted_iota(jnp.int32, (1, G, G), 2).astype(jnp.float32)

    ex = jnp.exp(top)
    sig8 = sig[0:8]

    aw = jnp.where(a == 0, anchors[0, 0], jnp.where(a == 1, anchors[1, 0], anchors[2, 0]))
    ah = jnp.where(a == 0, anchors[0, 1], jnp.where(a == 1, anchors[1, 1], anchors[2, 1]))

    base = jnp.where((row8 == 2) | (row8 == 3), ex, sig8)
    add = jnp.where(row8 == 0, gx, jnp.where(row8 == 1, gy, 0.0))
    scale = jnp.where(
        row8 < 2, stride, jnp.where(row8 == 2, aw, jnp.where(row8 == 3, ah, 1.0))
    )
    top_out = (base + add) * scale

    y = jnp.concatenate([top_out, sig[8:]], axis=0)

    for gyi in range(G):
        obuf[slot, gyi * G : (gyi + 1) * G, :] = y[:, gyi, :].T

    pltpu.make_async_copy(
        obuf.at[slot], o_hbm.at[b, pl.ds(a * GG, GG), :], osem.at[slot]
    ).start()

    @pl.when(i + _NBUF < nsteps)
    def _prefetch():
        bn = jax.lax.div(i + _NBUF, nA)
        an = jax.lax.rem(i + _NBUF, nA)
        pltpu.make_async_copy(
            x_hbm.at[bn, pl.ds(an * nch, nch)], ibuf.at[slot], isem.at[slot]
        ).start()

    @pl.when(i == nsteps - 1)
    def _drain():
        for k in range(_NBUF):
            pltpu.make_async_copy(
                obuf.at[k], o_hbm.at[0, pl.ds(0, GG), :], osem.at[k]
            ).wait()


def kernel(x):
    B = x.shape[0]
    G = x.shape[2]
    nA = _ANCHORS.shape[0]
    nch = _NUM_CLASSES + 5
    GG = G * G
    stride = _IMG_DIM / G
    nsteps = B * nA

    out = pl.pallas_call(
        functools.partial(
            _decode_kernel, G=G, stride=stride, anchors=_ANCHORS, nsteps=nsteps
        ),
        grid=(nsteps,),
        in_specs=[pl.BlockSpec(memory_space=pltpu.MemorySpace.HBM)],
        out_specs=pl.BlockSpec(memory_space=pltpu.MemorySpace.HBM),
        out_shape=jax.ShapeDtypeStruct((B, nA * GG, nch), jnp.float32),
        scratch_shapes=[
            pltpu.VMEM((_NBUF, nch, G, G), jnp.float32),
            pltpu.VMEM((_NBUF, GG, nch), jnp.float32),
            pltpu.SemaphoreType.DMA((_NBUF,)),
            pltpu.SemaphoreType.DMA((_NBUF,)),
        ],
        compiler_params=pltpu.CompilerParams(
            dimension_semantics=("arbitrary",),
        ),
    )(x)

    return out
